# Initial kernel scaffold; baseline (speedup 1.0000x reference)
import jax
import jax.numpy as jnp
from jax.experimental import pallas as pl


def kernel(x0, dst, src, w0_w, w0_b, w1_w, w1_b):
    raise NotImplementedError("write your pallas kernel here")


# speedup vs baseline: 1.6502x; 1.6502x over previous
import functools

import jax
import jax.numpy as jnp
from jax import lax
from jax.experimental import pallas as pl
from jax.experimental.pallas import tpu as pltpu
import jax.experimental.pallas.tpu_sc as plsc

NC = 2
NS = 16
NW = NC * NS
L = 16


def _sc_build(N, E, D, NPB, CH, G, interpret=False):
    NPAD = NW * NPB
    CAP = G + 2 * L
    TRASH = NPB
    n_chunks = E // CH
    steps = CH // L

    mesh = plsc.VectorSubcoreMesh(
        core_axis_name="c", subcore_axis_name="s",
        num_cores=NC, num_subcores=NS)

    @functools.partial(
        pl.kernel,
        compiler_params=pltpu.CompilerParams(needs_layout_passes=False),
        out_type=(
            jax.ShapeDtypeStruct((NPAD, D), jnp.float32),
            jax.ShapeDtypeStruct((NPAD, D), jnp.float32),
            jax.ShapeDtypeStruct((NPAD,), jnp.float32),
        ),
        mesh=mesh,
        scratch_types=[
            pltpu.VMEM((NPB + 1, D), jnp.float32),
            pltpu.VMEM((NPB + 1, D), jnp.float32),
            pltpu.VMEM((NPB,), jnp.float32),
            pltpu.VMEM((CH,), jnp.int32),
            pltpu.VMEM((CH,), jnp.int32),
            pltpu.VMEM((CAP,), jnp.int32),
            pltpu.VMEM((CAP,), jnp.int32),
            pltpu.VMEM((G,), jnp.int32),
            pltpu.VMEM((G, D), jnp.float32),
            pltpu.SemaphoreType.DMA,
        ],
        interpret=interpret,
    )
    def sc_scatter(x_hbm, dst_hbm, src_hbm, sum_hbm, mx_hbm, cnt_hbm,
                   sum_v, mx_v, cnt_v, dst_c, src_c, pend_s, pend_l,
                   gidx, rows, sem):
        wid = lax.axis_index("s") * NC + lax.axis_index("c")
        lo = wid * NPB

        zeros = jnp.zeros((L,), jnp.float32)
        ninf = jnp.full((L,), -jnp.inf, jnp.float32)
        ones = jnp.ones((L,), jnp.float32)
        trash_v = jnp.full((L,), TRASH, jnp.int32)

        def init_row(r, _):
            for j in range(D // L):
                sum_v[r, pl.ds(L * j, L)] = zeros
                mx_v[r, pl.ds(L * j, L)] = ninf
            return 0
        lax.fori_loop(0, NPB + 1, init_row, 0)
        for j in range(NPB // L):
            cnt_v[pl.ds(L * j, L)] = zeros
        for j in range(CAP // L):
            pend_s[pl.ds(L * j, L)] = jnp.zeros((L,), jnp.int32)
            pend_l[pl.ds(L * j, L)] = trash_v

        def flush():
            for j in range(G // L):
                gidx[pl.ds(L * j, L)] = pend_s[pl.ds(L * j, L)]
            pltpu.async_copy(x_hbm.at[gidx], rows, sem).wait()

            def acc(g, _):
                loc = pend_l[pl.ds(g, L)][0]
                for j in range(D // L):
                    v = rows[g, pl.ds(L * j, L)]
                    plsc.addupdate(sum_v.at[loc, pl.ds(L * j, L)], v)
                    mx_v[loc, pl.ds(L * j, L)] = jnp.maximum(
                        mx_v[loc, pl.ds(L * j, L)], v)
                return 0
            lax.fori_loop(0, G, acc, 0)

        def chunk_body(c, npend):
            pltpu.sync_copy(dst_hbm.at[pl.ds(c * CH, CH)], dst_c)
            pltpu.sync_copy(src_hbm.at[pl.ds(c * CH, CH)], src_c)

            def step(i, npend):
                d = dst_c[pl.ds(i * L, L)]
                s = src_c[pl.ds(i * L, L)]
                dl = d - lo
                mask = (dl >= 0) & (dl < NPB)
                m = jnp.sum(mask.astype(jnp.int32))

                @pl.when(m > 0)
                def _():
                    plsc.store_compressed(
                        pend_s.at[pl.ds(npend, L)], s, mask=mask)
                    plsc.store_compressed(
                        pend_l.at[pl.ds(npend, L)], dl, mask=mask)
                    plsc.addupdate_scatter(cnt_v, [dl], ones, mask=mask)

                npend = npend + m

                @pl.when(npend >= G)
                def _():
                    flush()
                    rs = pend_s[pl.ds(G, L)]
                    rl = pend_l[pl.ds(G, L)]
                    pend_s[pl.ds(0, L)] = rs
                    pend_l[pl.ds(0, L)] = rl

                return jnp.where(npend >= G, npend - G, npend)

            return lax.fori_loop(0, steps, step, npend)

        npend = lax.fori_loop(0, n_chunks, chunk_body, 0)

        iota = lax.iota(jnp.int32, L)
        for j in range(G // L):
            idxv = iota + L * j
            v = pend_l[pl.ds(L * j, L)]
            pend_l[pl.ds(L * j, L)] = jnp.where(idxv < npend, v, trash_v)
        flush()

        pltpu.sync_copy(sum_v.at[pl.ds(0, NPB)], sum_hbm.at[pl.ds(lo, NPB)])
        pltpu.sync_copy(mx_v.at[pl.ds(0, NPB)], mx_hbm.at[pl.ds(lo, NPB)])
        pltpu.sync_copy(cnt_v, cnt_hbm.at[pl.ds(lo, NPB)])

    return sc_scatter


def _tc_dense(x2d, ssum, smax, cnt2d, w0m, w0a, b0, w1, b1, R, interpret=False):
    N, D = x2d.shape
    grid = N // R

    def body(x_ref, s_ref, m_ref, c_ref, w0m_ref, w0a_ref, b0_ref,
             w1_ref, b1_ref, o_ref):
        x = x_ref[...]
        c = c_ref[...]
        has = c > 0.0
        mean = jnp.where(has, s_ref[...] / jnp.maximum(c, 1.0), x)
        amax = jnp.where(has, m_ref[...], x)
        h = lax.dot_general(mean, w0m_ref[...], (((1,), (1,)), ((), ())),
                            preferred_element_type=jnp.float32)
        h = h + lax.dot_general(amax, w0a_ref[...], (((1,), (1,)), ((), ())),
                                preferred_element_type=jnp.float32)
        h = jnp.maximum(h + b0_ref[...], 0.0)
        y = lax.dot_general(h, w1_ref[...], (((1,), (1,)), ((), ())),
                            preferred_element_type=jnp.float32)
        o_ref[...] = x + y + b1_ref[...]

    full = lambda i: (0, 0)
    return pl.pallas_call(
        body,
        grid=(grid,),
        in_specs=[
            pl.BlockSpec((R, D), lambda i: (i, 0)),
            pl.BlockSpec((R, D), lambda i: (i, 0)),
            pl.BlockSpec((R, D), lambda i: (i, 0)),
            pl.BlockSpec((R, 1), lambda i: (i, 0)),
            pl.BlockSpec((D, D), full),
            pl.BlockSpec((D, D), full),
            pl.BlockSpec((1, D), full),
            pl.BlockSpec((D, D), full),
            pl.BlockSpec((1, D), full),
        ],
        out_specs=pl.BlockSpec((R, D), lambda i: (i, 0)),
        out_shape=jax.ShapeDtypeStruct((N, D), jnp.float32),
        interpret=interpret,
    )(x2d, ssum, smax, cnt2d, w0m, w0a, b0, w1, b1)


def _run(x0, dst, src, w0_w, w0_b, w1_w, w1_b, interpret=False):
    B, N, D = x0.shape
    E = dst.shape[0]
    x2d = x0.reshape(N, D)
    dst32 = dst.astype(jnp.int32)
    src32 = src.astype(jnp.int32)

    NPB = (-(-N // NW) + 7) // 8 * 8
    CH = 2000 if E % 2000 == 0 else L
    G = 128

    sc = _sc_build(N, E, D, NPB, CH, G, interpret=interpret)
    ssum, smax, cnt = sc(x2d, dst32, src32)

    R = N // 5 if (N % 5 == 0 and (N // 5) % 8 == 0) else N
    out2d = _tc_dense(x2d, ssum[:, :], smax[:, :], cnt.reshape(-1, 1),
                      w0_w[:, :D], w0_w[:, D:], w0_b.reshape(1, D),
                      w1_w, w1_b.reshape(1, D), R, interpret=interpret)
    return out2d.reshape(B, N, D)


def kernel(x0, dst, src, w0_w, w0_b, w1_w, w1_b):
    return _run(x0, dst, src, w0_w, w0_b, w1_w, w1_b)


# speedup vs baseline: 2.1006x; 1.2730x over previous
import functools

import jax
import jax.numpy as jnp
from jax import lax
from jax.experimental import pallas as pl
from jax.experimental.pallas import tpu as pltpu
import jax.experimental.pallas.tpu_sc as plsc

NC = 2
NS = 16
NW = NC * NS
L = 16


def _sc_build(N, E, D, NPB, CH, G):
    NPAD = NW * NPB
    CAP = G + 2 * L
    TRASH = NPB
    n_chunks = E // CH
    steps = CH // L

    mesh = plsc.VectorSubcoreMesh(
        core_axis_name="c", subcore_axis_name="s",
        num_cores=NC, num_subcores=NS)

    @functools.partial(
        pl.kernel,
        compiler_params=pltpu.CompilerParams(
            needs_layout_passes=False, use_tc_tiling_on_sc=False),
        out_type=(
            jax.ShapeDtypeStruct((NPAD, D), jnp.float32),
            jax.ShapeDtypeStruct((NPAD, D), jnp.float32),
            jax.ShapeDtypeStruct((NPAD,), jnp.float32),
        ),
        mesh=mesh,
        scratch_types=[
            pltpu.VMEM((NPB + 1, D), jnp.float32),
            pltpu.VMEM((NPB + 1, D), jnp.float32),
            pltpu.VMEM((NPB,), jnp.float32),
            pltpu.VMEM((2, CH), jnp.int32),
            pltpu.VMEM((2, CH), jnp.int32),
            pltpu.VMEM((CAP,), jnp.int32),
            pltpu.VMEM((CAP,), jnp.int32),
            pltpu.VMEM((2, G), jnp.int32),
            pltpu.VMEM((2, G), jnp.int32),
            pltpu.VMEM((2, G, D), jnp.float32),
            pltpu.SemaphoreType.DMA,
            pltpu.SemaphoreType.DMA,
            pltpu.SemaphoreType.DMA,
            pltpu.SemaphoreType.DMA,
        ],
    )
    def sc_scatter(x_hbm, dst_hbm, src_hbm, sum_hbm, mx_hbm, cnt_hbm,
                   sum_v, mx_v, cnt_v, dst_c, src_c, pend_s, pend_l,
                   gidx, gloc, rows, semc0, semc1, semg0, semg1):
        wid = lax.axis_index("s") * NC + lax.axis_index("c")
        lo = wid * NPB

        zeros = jnp.zeros((L,), jnp.float32)
        ninf = jnp.full((L,), -jnp.inf, jnp.float32)
        ones = jnp.ones((L,), jnp.float32)
        trash_v = jnp.full((L,), TRASH, jnp.int32)
        semc = (semc0, semc1)
        semg = (semg0, semg1)

        def init_row(r, _):
            for j in range(D // L):
                sum_v[r, pl.ds(L * j, L)] = zeros
                mx_v[r, pl.ds(L * j, L)] = ninf
            return 0
        lax.fori_loop(0, NPB + 1, init_row, 0)
        for j in range(NPB // L):
            cnt_v[pl.ds(L * j, L)] = zeros
        for j in range(CAP // L):
            pend_s[pl.ds(L * j, L)] = jnp.zeros((L,), jnp.int32)
            pend_l[pl.ds(L * j, L)] = trash_v

        def fire(p):
            for j in range(G // L):
                gidx[p, pl.ds(L * j, L)] = pend_s[pl.ds(L * j, L)]
                gloc[p, pl.ds(L * j, L)] = pend_l[pl.ds(L * j, L)]
            return pltpu.async_copy(x_hbm.at[gidx.at[p]], rows.at[p], semg[p])

        def drain(p):
            pltpu.make_async_copy(
                x_hbm.at[pl.ds(0, G)], rows.at[p], semg[p]).wait()

            def acc(g, _):
                loc = gloc[p, pl.ds(g, L)][0]
                for j in range(D // L):
                    v = rows[p, g, pl.ds(L * j, L)]
                    plsc.addupdate(sum_v.at[loc, pl.ds(L * j, L)], v)
                    mx_v[loc, pl.ds(L * j, L)] = jnp.maximum(
                        mx_v[loc, pl.ds(L * j, L)], v)
                return 0
            lax.fori_loop(0, G, acc, 0)

        def scan_chunk(par, carry):
            def step(i, carry):
                npend, nfired = carry
                d = dst_c[par, pl.ds(i * L, L)]
                s = src_c[par, pl.ds(i * L, L)]
                dl = d - lo
                mask = (dl >= 0) & (dl < NPB)
                m = plsc.all_reduce_population_count(mask)[0]

                @pl.when(m > 0)
                def _():
                    plsc.store_compressed(
                        pend_s.at[pl.ds(npend, L)], s, mask=mask)
                    plsc.store_compressed(
                        pend_l.at[pl.ds(npend, L)], dl, mask=mask)
                    plsc.addupdate_scatter(cnt_v, [dl], ones, mask=mask)

                npend = npend + m
                flushed = npend >= G

                @pl.when(flushed)
                def _():
                    for p in (0, 1):
                        @pl.when(nfired % 2 == p)
                        def _():
                            fire(p)
                            @pl.when(nfired > 0)
                            def _():
                                drain(1 - p)
                    rs = pend_s[pl.ds(G, L)]
                    rl = pend_l[pl.ds(G, L)]
                    pend_s[pl.ds(0, L)] = rs
                    pend_l[pl.ds(0, L)] = rl

                npend = jnp.where(flushed, npend - G, npend)
                nfired = jnp.where(flushed, nfired + 1, nfired)
                return npend, nfired
            return lax.fori_loop(0, steps, step, carry)

        pltpu.async_copy(dst_hbm.at[pl.ds(0, CH)], dst_c.at[0], semc0)
        pltpu.async_copy(src_hbm.at[pl.ds(0, CH)], src_c.at[0], semc0)

        def pair_body(pair, carry):
            for par in (0, 1):
                c = pair * 2 + par

                @pl.when(c + 1 < n_chunks)
                def _():
                    pltpu.async_copy(
                        dst_hbm.at[pl.ds((c + 1) * CH, CH)],
                        dst_c.at[1 - par], semc[1 - par])
                    pltpu.async_copy(
                        src_hbm.at[pl.ds((c + 1) * CH, CH)],
                        src_c.at[1 - par], semc[1 - par])

                pltpu.make_async_copy(
                    dst_hbm.at[pl.ds(0, CH)], dst_c.at[par], semc[par]).wait()
                pltpu.make_async_copy(
                    src_hbm.at[pl.ds(0, CH)], src_c.at[par], semc[par]).wait()
                carry = scan_chunk(par, carry)
            return carry

        npend, nfired = lax.fori_loop(0, n_chunks // 2, pair_body, (0, 0))

        for p in (0, 1):
            @pl.when((nfired > 0) & (nfired % 2 == 1 - p))
            def _():
                drain(p)

        iota = lax.iota(jnp.int32, L)
        for j in range(G // L):
            idxv = iota + L * j
            v = pend_l[pl.ds(L * j, L)]
            pend_l[pl.ds(L * j, L)] = jnp.where(idxv < npend, v, trash_v)
        for p in (0, 1):
            @pl.when(nfired % 2 == p)
            def _():
                fire(p)
                drain(p)

        pltpu.sync_copy(sum_v.at[pl.ds(0, NPB)], sum_hbm.at[pl.ds(lo, NPB)])
        pltpu.sync_copy(mx_v.at[pl.ds(0, NPB)], mx_hbm.at[pl.ds(lo, NPB)])
        pltpu.sync_copy(cnt_v, cnt_hbm.at[pl.ds(lo, NPB)])

    return sc_scatter


def _tc_dense(x2d, ssum, smax, cnt2d, w0m, w0a, b0, w1, b1, R):
    N, D = x2d.shape
    grid = N // R

    def body(x_ref, s_ref, m_ref, c_ref, w0m_ref, w0a_ref, b0_ref,
             w1_ref, b1_ref, o_ref):
        x = x_ref[...]
        c = c_ref[...]
        has = c > 0.0
        mean = jnp.where(has, s_ref[...] / jnp.maximum(c, 1.0), x)
        amax = jnp.where(has, m_ref[...], x)
        h = lax.dot_general(mean, w0m_ref[...], (((1,), (1,)), ((), ())),
                            preferred_element_type=jnp.float32)
        h = h + lax.dot_general(amax, w0a_ref[...], (((1,), (1,)), ((), ())),
                                preferred_element_type=jnp.float32)
        h = jnp.maximum(h + b0_ref[...], 0.0)
        y = lax.dot_general(h, w1_ref[...], (((1,), (1,)), ((), ())),
                            preferred_element_type=jnp.float32)
        o_ref[...] = x + y + b1_ref[...]

    full = lambda i: (0, 0)
    return pl.pallas_call(
        body,
        grid=(grid,),
        in_specs=[
            pl.BlockSpec((R, D), lambda i: (i, 0)),
            pl.BlockSpec((R, D), lambda i: (i, 0)),
            pl.BlockSpec((R, D), lambda i: (i, 0)),
            pl.BlockSpec((R, 1), lambda i: (i, 0)),
            pl.BlockSpec((D, D), full),
            pl.BlockSpec((D, D), full),
            pl.BlockSpec((1, D), full),
            pl.BlockSpec((D, D), full),
            pl.BlockSpec((1, D), full),
        ],
        out_specs=pl.BlockSpec((R, D), lambda i: (i, 0)),
        out_shape=jax.ShapeDtypeStruct((N, D), jnp.float32),
    )(x2d, ssum, smax, cnt2d, w0m, w0a, b0, w1, b1)


def kernel(x0, dst, src, w0_w, w0_b, w1_w, w1_b):
    B, N, D = x0.shape
    E = dst.shape[0]
    x2d = x0.reshape(N, D)
    dst32 = dst.astype(jnp.int32)
    src32 = src.astype(jnp.int32)

    NPB = (-(-N // NW) + 7) // 8 * 8
    CH = 2000 if E % 2000 == 0 else L
    G = 128

    sc = _sc_build(N, E, D, NPB, CH, G)
    ssum, smax, cnt = sc(x2d, dst32, src32)

    R = N // 5 if (N % 5 == 0 and (N // 5) % 8 == 0) else N
    out2d = _tc_dense(x2d, ssum, smax, cnt.reshape(-1, 1),
                      w0_w[:, :D], w0_w[:, D:], w0_b.reshape(1, D),
                      w1_w, w1_b.reshape(1, D), R)
    return out2d.reshape(B, N, D)


# speedup vs baseline: 2.5648x; 1.2210x over previous
import functools

import jax
import jax.numpy as jnp
from jax import lax
from jax.experimental import pallas as pl
from jax.experimental.pallas import tpu as pltpu
import jax.experimental.pallas.tpu_sc as plsc

NC = 2
NS = 16
NW = NC * NS
L = 16


def _sc_build(N, E, D, NPB, CH, G):
    NPAD = NW * NPB
    CAP = G + 2 * L
    TRASH = NPB
    n_chunks = E // CH
    steps = CH // (2 * L)

    mesh = plsc.VectorSubcoreMesh(
        core_axis_name="c", subcore_axis_name="s",
        num_cores=NC, num_subcores=NS)

    @functools.partial(
        pl.kernel,
        compiler_params=pltpu.CompilerParams(
            needs_layout_passes=False, use_tc_tiling_on_sc=False),
        out_type=(
            jax.ShapeDtypeStruct((NPAD, D), jnp.float32),
            jax.ShapeDtypeStruct((NPAD, D), jnp.float32),
            jax.ShapeDtypeStruct((NPAD,), jnp.float32),
        ),
        mesh=mesh,
        scratch_types=[
            pltpu.VMEM((NPB + 1, D), jnp.float32),
            pltpu.VMEM((NPB + 1, D), jnp.float32),
            pltpu.VMEM((NPB,), jnp.float32),
            pltpu.VMEM((2, CH), jnp.int32),
            pltpu.VMEM((2, CH), jnp.int32),
            pltpu.VMEM((CAP,), jnp.int32),
            pltpu.VMEM((CAP,), jnp.int32),
            pltpu.VMEM((2, G), jnp.int32),
            pltpu.VMEM((2, G), jnp.int32),
            pltpu.VMEM((2, G, D), jnp.float32),
            pltpu.SemaphoreType.DMA,
            pltpu.SemaphoreType.DMA,
            pltpu.SemaphoreType.DMA,
            pltpu.SemaphoreType.DMA,
        ],
    )
    def sc_scatter(x_hbm, dst_hbm, src_hbm, sum_hbm, mx_hbm, cnt_hbm,
                   sum_v, mx_v, cnt_v, dst_c, src_c, pend_s, pend_l,
                   gidx, gloc, rows, semc0, semc1, semg0, semg1):
        wid = lax.axis_index("s") * NC + lax.axis_index("c")
        lo = wid * NPB

        zeros = jnp.zeros((L,), jnp.float32)
        ninf = jnp.full((L,), -jnp.inf, jnp.float32)
        ones = jnp.ones((L,), jnp.float32)
        trash_v = jnp.full((L,), TRASH, jnp.int32)
        semc = (semc0, semc1)
        semg = (semg0, semg1)

        def init_row(r, _):
            for j in range(D // L):
                sum_v[r, pl.ds(L * j, L)] = zeros
                mx_v[r, pl.ds(L * j, L)] = ninf
            return 0
        lax.fori_loop(0, NPB + 1, init_row, 0)
        for j in range(NPB // L):
            cnt_v[pl.ds(L * j, L)] = zeros
        for j in range(CAP // L):
            pend_s[pl.ds(L * j, L)] = jnp.zeros((L,), jnp.int32)
            pend_l[pl.ds(L * j, L)] = trash_v

        def fire(p):
            for j in range(G // L):
                gidx[p, pl.ds(L * j, L)] = pend_s[pl.ds(L * j, L)]
                gloc[p, pl.ds(L * j, L)] = pend_l[pl.ds(L * j, L)]
            return pltpu.async_copy(x_hbm.at[gidx.at[p]], rows.at[p], semg[p])

        def drain(p):
            pltpu.make_async_copy(
                x_hbm.at[pl.ds(0, G)], rows.at[p], semg[p]).wait()

            def acc(gg, _):
                locv = gloc[p, pl.ds(gg * L, L)]
                for k in range(L):
                    loc = locv[k]
                    g = gg * L + k
                    for j in range(D // L):
                        v = rows[p, g, pl.ds(L * j, L)]
                        plsc.addupdate(sum_v.at[loc, pl.ds(L * j, L)], v)
                        mx_v[loc, pl.ds(L * j, L)] = jnp.maximum(
                            mx_v[loc, pl.ds(L * j, L)], v)
                return 0
            lax.fori_loop(0, G // L, acc, 0)

        def scan_chunk(par, carry):
            def step(i, carry):
                npend, nfired = carry
                for half in (0, 1):
                    d = dst_c[par, pl.ds(i * 2 * L + half * L, L)]
                    s = src_c[par, pl.ds(i * 2 * L + half * L, L)]
                    dl = d - lo
                    mask = (dl >= 0) & (dl < NPB)
                    m = plsc.all_reduce_population_count(mask)[0]

                    @pl.when(m > 0)
                    def _():
                        plsc.store_compressed(
                            pend_s.at[pl.ds(npend, L)], s, mask=mask)
                        plsc.store_compressed(
                            pend_l.at[pl.ds(npend, L)], dl, mask=mask)
                        plsc.addupdate_scatter(cnt_v, [dl], ones, mask=mask)

                    npend = npend + m

                flushed = npend >= G

                @pl.when(flushed)
                def _():
                    for p in (0, 1):
                        @pl.when(nfired % 2 == p)
                        def _():
                            fire(p)
                            @pl.when(nfired > 0)
                            def _():
                                drain(1 - p)
                    for j in (0, 1):
                        rs = pend_s[pl.ds(G + L * j, L)]
                        rl = pend_l[pl.ds(G + L * j, L)]
                        pend_s[pl.ds(L * j, L)] = rs
                        pend_l[pl.ds(L * j, L)] = rl

                npend = jnp.where(flushed, npend - G, npend)
                nfired = jnp.where(flushed, nfired + 1, nfired)
                return npend, nfired
            return lax.fori_loop(0, steps, step, carry)

        pltpu.async_copy(dst_hbm.at[pl.ds(0, CH)], dst_c.at[0], semc0)
        pltpu.async_copy(src_hbm.at[pl.ds(0, CH)], src_c.at[0], semc0)

        def pair_body(pair, carry):
            for par in (0, 1):
                c = pair * 2 + par

                @pl.when(c + 1 < n_chunks)
                def _():
                    pltpu.async_copy(
                        dst_hbm.at[pl.ds((c + 1) * CH, CH)],
                        dst_c.at[1 - par], semc[1 - par])
                    pltpu.async_copy(
                        src_hbm.at[pl.ds((c + 1) * CH, CH)],
                        src_c.at[1 - par], semc[1 - par])

                pltpu.make_async_copy(
                    dst_hbm.at[pl.ds(0, CH)], dst_c.at[par], semc[par]).wait()
                pltpu.make_async_copy(
                    src_hbm.at[pl.ds(0, CH)], src_c.at[par], semc[par]).wait()
                carry = scan_chunk(par, carry)
            return carry

        npend, nfired = lax.fori_loop(0, n_chunks // 2, pair_body, (0, 0))

        for p in (0, 1):
            @pl.when((nfired > 0) & (nfired % 2 == 1 - p))
            def _():
                drain(p)

        iota = lax.iota(jnp.int32, L)
        for j in range(G // L):
            idxv = iota + L * j
            v = pend_l[pl.ds(L * j, L)]
            pend_l[pl.ds(L * j, L)] = jnp.where(idxv < npend, v, trash_v)
        for p in (0, 1):
            @pl.when(nfired % 2 == p)
            def _():
                fire(p)
                drain(p)

        pltpu.sync_copy(sum_v.at[pl.ds(0, NPB)], sum_hbm.at[pl.ds(lo, NPB)])
        pltpu.sync_copy(mx_v.at[pl.ds(0, NPB)], mx_hbm.at[pl.ds(lo, NPB)])
        pltpu.sync_copy(cnt_v, cnt_hbm.at[pl.ds(lo, NPB)])

    return sc_scatter


def _tc_dense(x2d, ssum, smax, cnt2d, w0m, w0a, b0, w1, b1, R):
    N, D = x2d.shape
    grid = N // R

    def body(x_ref, s_ref, m_ref, c_ref, w0m_ref, w0a_ref, b0_ref,
             w1_ref, b1_ref, o_ref):
        x = x_ref[...]
        c = c_ref[...]
        has = c > 0.0
        mean = jnp.where(has, s_ref[...] / jnp.maximum(c, 1.0), x)
        amax = jnp.where(has, m_ref[...], x)
        h = lax.dot_general(mean, w0m_ref[...], (((1,), (1,)), ((), ())),
                            preferred_element_type=jnp.float32)
        h = h + lax.dot_general(amax, w0a_ref[...], (((1,), (1,)), ((), ())),
                                preferred_element_type=jnp.float32)
        h = jnp.maximum(h + b0_ref[...], 0.0)
        y = lax.dot_general(h, w1_ref[...], (((1,), (1,)), ((), ())),
                            preferred_element_type=jnp.float32)
        o_ref[...] = x + y + b1_ref[...]

    full = lambda i: (0, 0)
    return pl.pallas_call(
        body,
        grid=(grid,),
        in_specs=[
            pl.BlockSpec((R, D), lambda i: (i, 0)),
            pl.BlockSpec((R, D), lambda i: (i, 0)),
            pl.BlockSpec((R, D), lambda i: (i, 0)),
            pl.BlockSpec((R, 1), lambda i: (i, 0)),
            pl.BlockSpec((D, D), full),
            pl.BlockSpec((D, D), full),
            pl.BlockSpec((1, D), full),
            pl.BlockSpec((D, D), full),
            pl.BlockSpec((1, D), full),
        ],
        out_specs=pl.BlockSpec((R, D), lambda i: (i, 0)),
        out_shape=jax.ShapeDtypeStruct((N, D), jnp.float32),
    )(x2d, ssum, smax, cnt2d, w0m, w0a, b0, w1, b1)


def kernel(x0, dst, src, w0_w, w0_b, w1_w, w1_b):
    B, N, D = x0.shape
    E = dst.shape[0]
    x2d = x0.reshape(N, D)
    dst32 = dst.astype(jnp.int32)
    src32 = src.astype(jnp.int32)

    NPB = (-(-N // NW) + 7) // 8 * 8
    CH = 1600 if E % 1600 == 0 else 2 * L
    G = 128

    sc = _sc_build(N, E, D, NPB, CH, G)
    ssum, smax, cnt = sc(x2d, dst32, src32)

    R = N // 5 if (N % 5 == 0 and (N // 5) % 8 == 0) else N
    out2d = _tc_dense(x2d, ssum, smax, cnt.reshape(-1, 1),
                      w0_w[:, :D], w0_w[:, D:], w0_b.reshape(1, D),
                      w1_w, w1_b.reshape(1, D), R)
    return out2d.reshape(B, N, D)
